# Initial kernel scaffold; baseline (speedup 1.0000x reference)
import jax
import jax.numpy as jnp
from jax.experimental import pallas as pl


def kernel(x, edge_index, pp, W1, Wc1, Wc2, Wc3, b1, b2, b3, Wp):
    raise NotImplementedError("write your pallas kernel here")


# speedup vs baseline: 4.1796x; 4.1796x over previous
import functools
import math

import jax
import jax.numpy as jnp
from jax import lax
from jax.experimental import pallas as pl
from jax.experimental.pallas import tpu as pltpu
from jax.experimental.pallas import tpu_sc as plsc

N_NODES = 10000
N_EDGES = 160000
IN_FEATS = 128
N_HIDDEN = 256
N_CLASSES = 64
ALPHA = 0.1
LAMBDA = 1.0

NC = 2
NS = 16
NPAD = 10240
EPS = N_EDGES // NS
EB = 128
NCHUNK = EPS // EB
ETAIL = EPS - NCHUNK * EB
RPS = NPAD // NS
ZR = 128

def _sc_mesh():
    return plsc.VectorSubcoreMesh(core_axis_name="c", subcore_axis_name="s",
                                  num_cores=NC, num_subcores=NS)


def _zero_rows(zbuf, nrows, ncols):
    def row(i, _):
        for j in range(ncols // 16):
            zbuf[i, pl.ds(j * 16, 16)] = jnp.zeros((16,), jnp.float32)
        return 0
    lax.fori_loop(0, nrows, row, 0)


@functools.cache
def _degrees_kernel():
    return pl.kernel(
        _degrees,
        out_type=[
            jax.ShapeDtypeStruct((NPAD, 16), jnp.float32),
            jax.ShapeDtypeStruct((NPAD, 16), jnp.float32),
        ],
        mesh=_sc_mesh(),
        compiler_params=pltpu.CompilerParams(needs_layout_passes=False),
        scratch_types=[
            pltpu.VMEM((EPS,), jnp.int32),
            pltpu.VMEM((NPAD,), jnp.float32),
            pltpu.VMEM((NS, NPAD // NS), jnp.float32),
            pltpu.VMEM((NPAD // NS, 16), jnp.float32),
            pltpu.VMEM_SHARED((NS, NPAD), jnp.float32),
        ],
    )


def _degrees(src_hbm, dst_hbm, dego_hbm, degi_hbm, idxb, hist, mslab, stage,
             sh):
    c = lax.axis_index("c")
    s = lax.axis_index("s")
    mcols = NPAD // NS
    zero16 = jnp.zeros((16,), jnp.float32)
    one16 = jnp.ones((16,), jnp.float32)

    def zrow(i, _):
        hist[pl.ds(i * 16, 16)] = zero16
        return 0
    lax.fori_loop(0, NPAD // 16, zrow, 0)

    @pl.when(c == 0)
    def _():
        pltpu.sync_copy(src_hbm.at[pl.ds(s * EPS, EPS)], idxb)

    @pl.when(c == 1)
    def _():
        pltpu.sync_copy(dst_hbm.at[pl.ds(s * EPS, EPS)], idxb)

    def grp(g, _):
        iv = idxb[pl.ds(g * 16, 16)]
        plsc.addupdate_scatter(hist, [iv], one16)
        return 0
    lax.fori_loop(0, EPS // 16, grp, 0)

    pltpu.sync_copy(hist, sh.at[s])
    plsc.subcore_barrier()

    pltpu.sync_copy(sh.at[:, pl.ds(s * mcols, mcols)], mslab)
    rows0 = lax.iota(jnp.int32, 16)
    col0 = jnp.zeros((16,), jnp.int32)

    def red(t, _):
        acc = zero16
        for r in range(NS):
            acc = acc + mslab[r, pl.ds(t * 16, 16)]
        plsc.store_scatter(stage, [rows0 + t * 16, col0], acc)
        return 0
    lax.fori_loop(0, mcols // 16, red, 0)

    @pl.when(c == 0)
    def _():
        pltpu.sync_copy(stage, dego_hbm.at[pl.ds(s * mcols, mcols)])

    @pl.when(c == 1)
    def _():
        pltpu.sync_copy(stage, degi_hbm.at[pl.ds(s * mcols, mcols)])


@functools.cache
def _spmm_kernel():
    return pl.kernel(
        _spmm,
        out_type=[
            jax.ShapeDtypeStruct((NPAD, 128), jnp.float32),
            jax.ShapeDtypeStruct((NPAD, 128), jnp.float32),
        ],
        mesh=_sc_mesh(),
        scratch_types=[
            pltpu.VMEM((EB,), jnp.int32),
            pltpu.VMEM((EB,), jnp.int32),
            pltpu.VMEM((EB,), jnp.int32),
            pltpu.VMEM((EB, 128), jnp.float32),
            pltpu.VMEM((ZR, 128), jnp.float32),
            pltpu.SemaphoreType.DMA,
            pltpu.VMEM_SHARED((NPAD, 128), jnp.float32),
        ],
    )


def _spmm(hs_l, hs_r, src_hbm, dst_hbm, out_l, out_r, sidx, didx, tidx, rows,
          zbuf, sem, agg_sh):
    c = lax.axis_index("c")
    s = lax.axis_index("s")

    _zero_rows(zbuf, ZR, 128)
    for k in range(RPS // ZR):
        pltpu.sync_copy(zbuf, agg_sh.at[pl.ds(s * RPS + k * ZR, ZR)])
    plsc.subcore_barrier()

    ebase = s * EPS

    def chunk(i, _):
        off = ebase + i * EB
        pltpu.sync_copy(src_hbm.at[pl.ds(off, EB)], sidx)
        pltpu.sync_copy(dst_hbm.at[pl.ds(off, EB)], didx)

        @pl.when(c == 0)
        def _():
            pltpu.async_copy(hs_l.at[sidx], rows, sem).wait()

        @pl.when(c == 1)
        def _():
            pltpu.async_copy(hs_r.at[sidx], rows, sem).wait()

        pltpu.sync_copy(rows, agg_sh.at[didx], add=True)
        return 0

    lax.fori_loop(0, NCHUNK, chunk, 0)

    if ETAIL:
        toff = ebase + NCHUNK * EB
        pltpu.sync_copy(src_hbm.at[pl.ds(toff, ETAIL)],
                        tidx.at[pl.ds(0, ETAIL)])
        pltpu.sync_copy(dst_hbm.at[pl.ds(toff, ETAIL)],
                        didx.at[pl.ds(0, ETAIL)])

        @pl.when(c == 0)
        def _():
            pltpu.async_copy(hs_l.at[tidx.at[pl.ds(0, ETAIL)]],
                             rows.at[pl.ds(0, ETAIL)], sem).wait()

        @pl.when(c == 1)
        def _():
            pltpu.async_copy(hs_r.at[tidx.at[pl.ds(0, ETAIL)]],
                             rows.at[pl.ds(0, ETAIL)], sem).wait()

        pltpu.sync_copy(rows.at[pl.ds(0, ETAIL)],
                        agg_sh.at[didx.at[pl.ds(0, ETAIL)]], add=True)

    plsc.subcore_barrier()

    @pl.when(c == 0)
    def _():
        pltpu.sync_copy(agg_sh.at[pl.ds(s * RPS, RPS)],
                        out_l.at[pl.ds(s * RPS, RPS)])

    @pl.when(c == 1)
    def _():
        pltpu.sync_copy(agg_sh.at[pl.ds(s * RPS, RPS)],
                        out_r.at[pl.ds(s * RPS, RPS)])


_R = 1000
_GRID = (N_NODES // _R,)


def _norm(deg_blk):
    d = deg_blk[:, 0:1]
    return jnp.where(d > 0, lax.rsqrt(d), 0.0)


def _head_body(x_ref, pp_ref, w1_ref, dego_ref, x1_ref, sel_ref, hsl_ref,
               hsr_ref):
    x = x_ref[...]
    pp = pp_ref[...]
    sim = lax.dot_general(x, pp, (((1,), (1,)), ((), ())),
                          preferred_element_type=jnp.float32)
    mx = jnp.max(sim, axis=1, keepdims=True)
    cols = lax.broadcasted_iota(jnp.int32, sim.shape, 1)
    cand = jnp.where(sim == mx, cols, N_CLASSES)
    amin = jnp.min(cand, axis=1, keepdims=True)
    onehot = (cols == amin).astype(jnp.float32)
    sel = jnp.dot(onehot, pp, preferred_element_type=jnp.float32)
    w1 = w1_ref[...]
    x1 = (jnp.dot(x, w1[:IN_FEATS], preferred_element_type=jnp.float32)
          + jnp.dot(sel, w1[IN_FEATS:], preferred_element_type=jnp.float32))
    hs = x1 * _norm(dego_ref[...])
    x1_ref[...] = x1
    sel_ref[...] = sel
    hsl_ref[...] = hs[:, :128]
    hsr_ref[...] = hs[:, 128:]


def _mid_body(beta, rl_ref, rr_ref, x1_ref, degi_ref, dego_ref, w_ref, b_ref,
              hsl_ref, hsr_ref):
    agg = jnp.concatenate([rl_ref[...], rr_ref[...]], axis=1)
    agg = agg * _norm(degi_ref[...])
    rst = (1.0 - ALPHA) * agg + ALPHA * x1_ref[...]
    h = (rst * (1.0 - beta)
         + beta * jnp.dot(rst, w_ref[...], preferred_element_type=jnp.float32)
         + b_ref[...])
    hs = h * _norm(dego_ref[...])
    hsl_ref[...] = hs[:, :128]
    hsr_ref[...] = hs[:, 128:]


def _final_body(beta, rl_ref, rr_ref, x1_ref, degi_ref, w_ref, b_ref, sel_ref,
                wp_ref, out_ref):
    agg = jnp.concatenate([rl_ref[...], rr_ref[...]], axis=1)
    agg = agg * _norm(degi_ref[...])
    rst = (1.0 - ALPHA) * agg + ALPHA * x1_ref[...]
    h = (rst * (1.0 - beta)
         + beta * jnp.dot(rst, w_ref[...], preferred_element_type=jnp.float32)
         + b_ref[...])
    wp = wp_ref[...]
    out_ref[...] = (
        jnp.dot(jax.nn.relu(h), wp[:N_HIDDEN],
                preferred_element_type=jnp.float32)
        + jnp.dot(jax.nn.relu(sel_ref[...]), wp[N_HIDDEN:],
                  preferred_element_type=jnp.float32))


def _rowspec(cols):
    return pl.BlockSpec((_R, cols), lambda i: (i, 0))


def _fullspec(rows, cols):
    return pl.BlockSpec((rows, cols), lambda i: (0, 0))


_f32 = jnp.float32


def _head(x, pp, w1, dego):
    return pl.pallas_call(
        _head_body,
        grid=_GRID,
        in_specs=[_rowspec(IN_FEATS), _fullspec(N_CLASSES, IN_FEATS),
                  _fullspec(2 * IN_FEATS, N_HIDDEN), _rowspec(16)],
        out_specs=[_rowspec(N_HIDDEN), _rowspec(IN_FEATS),
                   _rowspec(128), _rowspec(128)],
        out_shape=[jax.ShapeDtypeStruct((N_NODES, N_HIDDEN), _f32),
                   jax.ShapeDtypeStruct((N_NODES, IN_FEATS), _f32),
                   jax.ShapeDtypeStruct((N_NODES, 128), _f32),
                   jax.ShapeDtypeStruct((N_NODES, 128), _f32)],
    )(x, pp, w1, dego)


def _mid(beta, rl, rr, x1, degi, dego, w, b):
    return pl.pallas_call(
        functools.partial(_mid_body, beta),
        grid=_GRID,
        in_specs=[_rowspec(128), _rowspec(128), _rowspec(N_HIDDEN),
                  _rowspec(16), _rowspec(16),
                  _fullspec(N_HIDDEN, N_HIDDEN), _fullspec(1, N_HIDDEN)],
        out_specs=[_rowspec(128), _rowspec(128)],
        out_shape=[jax.ShapeDtypeStruct((N_NODES, 128), _f32),
                   jax.ShapeDtypeStruct((N_NODES, 128), _f32)],
    )(rl, rr, x1, degi, dego, w, b)


def _final(beta, rl, rr, x1, degi, w, b, sel, wp):
    return pl.pallas_call(
        functools.partial(_final_body, beta),
        grid=_GRID,
        in_specs=[_rowspec(128), _rowspec(128), _rowspec(N_HIDDEN),
                  _rowspec(16), _fullspec(N_HIDDEN, N_HIDDEN),
                  _fullspec(1, N_HIDDEN), _rowspec(IN_FEATS),
                  _fullspec(N_HIDDEN + IN_FEATS, N_CLASSES)],
        out_specs=[_rowspec(N_CLASSES)],
        out_shape=[jax.ShapeDtypeStruct((N_NODES, N_CLASSES), _f32)],
    )(rl, rr, x1, degi, w, b, sel, wp)


def kernel(x, edge_index, pp, W1, Wc1, Wc2, Wc3, b1, b2, b3, Wp):
    src = edge_index[0].astype(jnp.int32)
    dst = edge_index[1].astype(jnp.int32)

    _DBG_JNP_DEG = False
    if _DBG_JNP_DEG:
        do = jnp.zeros((N_NODES,), jnp.float32).at[src].add(1.0)
        di = jnp.zeros((N_NODES,), jnp.float32).at[dst].add(1.0)
        dego = jnp.zeros((N_NODES, 16), jnp.float32).at[:, 0].set(do)
        degi = jnp.zeros((N_NODES, 16), jnp.float32).at[:, 0].set(di)
    else:
        dego, degi = _degrees_kernel()(src, dst)
        dego = dego[:N_NODES]
        degi = degi[:N_NODES]

    x1, sel, h1l, h1r = _head(x, pp, W1, dego)

    b1r = b1.reshape(1, N_HIDDEN)
    b2r = b2.reshape(1, N_HIDDEN)
    b3r = b3.reshape(1, N_HIDDEN)

    beta1 = math.log(LAMBDA / 1.0 + 1.0)
    beta2 = math.log(LAMBDA / 2.0 + 1.0)
    beta3 = math.log(LAMBDA / 3.0 + 1.0)

    r1l, r1r = _spmm_kernel()(h1l, h1r, src, dst)
    h2l, h2r = _mid(beta1, r1l[:N_NODES], r1r[:N_NODES], x1, degi, dego,
                    Wc1, b1r)

    r2l, r2r = _spmm_kernel()(h2l, h2r, src, dst)
    h3l, h3r = _mid(beta2, r2l[:N_NODES], r2r[:N_NODES], x1, degi, dego,
                    Wc2, b2r)

    r3l, r3r = _spmm_kernel()(h3l, h3r, src, dst)
    (out,) = _final(beta3, r3l[:N_NODES], r3r[:N_NODES], x1, degi, Wc3, b3r,
                    sel, Wp)

    return out
